# baseline (speedup 1.0000x reference)
import functools

import numpy as np
import jax
import jax.numpy as jnp
from jax import lax
from jax.experimental import pallas as pl
from jax.experimental.pallas import tpu as pltpu
from jax.experimental.pallas import tpu_sc as plsc

K = 256
ROWS = 128
COLS = 32768
NW = 32
ROWS_PER_W = ROWS // NW
NVREG_ROW = COLS // 16
KV = K // 16

_GDN = lax.GatherDimensionNumbers(
    offset_dims=(), collapsed_slice_dims=(0,), start_index_map=(0,)
)


def _perm(x, idx2d):
    return lax.gather(
        x, idx2d, _GDN, (1,),
        mode=lax.GatherScatterMode.PROMISE_IN_BOUNDS,
    )


def _key(x):
    i = lax.bitcast_convert_type(x, jnp.int32)
    return i ^ ((i >> 31) & jnp.int32(0x7FFFFFFF))


def _body(logits_hbm, vals_hbm, idxs_hbm,
          rowbuf, candk, candi, hist, wink, wini, obufv, obufi, dmasem):
    wid = lax.axis_index("s") * 2 + lax.axis_index("c")
    iota = lax.iota(jnp.int32, 16)
    ones = jnp.ones((16,), jnp.int32)
    zeros = jnp.zeros((16,), jnp.int32)
    lane_off = iota * 273
    lane_off128 = iota * 273 + 128
    idx15 = ((iota & 0) + 15).reshape(16, 1)

    def lastlane(x):
        return _perm(x, idx15)

    def clear_hist():
        @plsc.parallel_loop(0, 18, unroll=2)
        def cb(j):
            for l in range(16):
                hist[pl.ds(l * 273 + 16 * j, 16)] = zeros

    def bin_scan(kneed):
        hv = hist[pl.ds(256, 16)]
        for l in range(1, 16):
            hv = hv + hist[pl.ds(l * 273 + 256, 16)]
        cum = jnp.sum(hv)
        d_star = jnp.int32(256)
        cnt_gt = jnp.int32(0)
        found = jnp.int32(0)
        for j in range(15, -1, -1):
            totv = hist[pl.ds(16 * j, 16)]
            for l in range(1, 16):
                totv = totv + hist[pl.ds(l * 273 + 16 * j, 16)]
            sfx = lax.rev(lax.cumsum(lax.rev(totv, (0,)), axis=0), (0,)) + cum
            mask = sfx >= kneed
            popc = jnp.sum(mask.astype(jnp.int32))
            anyhit = (popc > 0).astype(jnp.int32)
            blane = popc - 1
            sfx_at = jnp.min(jnp.where(mask, sfx, jnp.int32(2**30)))
            tot_at = jnp.sum(jnp.where(iota == blane, totv, 0))
            first = anyhit * (1 - found)
            d_star = jnp.where(first > 0, 16 * j + blane, d_star)
            cnt_gt = jnp.where(first > 0, sfx_at - tot_at, cnt_gt)
            found = jnp.maximum(found, anyhit)
            cum = jnp.max(sfx)
        return d_star, cnt_gt

    def do_row(r, _):
        row = wid * ROWS_PER_W + r

        clear_hist()

        @plsc.parallel_loop(0, NVREG_ROW, unroll=8)
        def h1(i):
            t = lax.bitcast_convert_type(
                rowbuf[pl.ds(16 * i, 16)], jnp.int32) >> 24
            b = t ^ ((t >> 8) & jnp.int32(0x7F))
            plsc.addupdate_scatter(hist, [lane_off128 + b], ones)

        d0, _cg0 = bin_scan(jnp.int32(K))
        d0s = d0 - 128

        @plsc.parallel_loop(0, NVREG_ROW, unroll=16,
                            carry=jnp.zeros((16,), jnp.int32))
        def p1(i, offv):
            ik = _key(rowbuf[pl.ds(16 * i, 16)])
            m = (ik >> 24) >= d0s
            mi = m.astype(jnp.int32)
            cs = lax.cumsum(mi, axis=0)
            pos = offv + cs - mi
            plsc.store_scatter(candk, [pos], ik, mask=m)
            plsc.store_scatter(candi, [pos], iota + 16 * i, mask=m)
            return offv + lastlane(cs)
        cand_n = p1[0]

        nxt = jnp.minimum(r + 1, ROWS_PER_W - 1)
        cp = pltpu.async_copy(
            logits_hbm.at[wid * ROWS_PER_W + nxt], rowbuf, dmasem)

        def do_round(t, carry):
            cand_n, kneed, p_hi, woffv = carry
            s = 16 - 8 * t
            clear_hist()
            nv = (cand_n + 15) // 16

            @plsc.parallel_loop(0, nv, unroll=4)
            def hb(i):
                ik = candk[pl.ds(16 * i, 16)]
                valid = (16 * i + iota) < cand_n
                b = jnp.where((ik >> (s + 8)) == p_hi,
                              (ik >> s) & 0xFF, jnp.int32(256))
                plsc.addupdate_scatter(hist, [lane_off + b], ones, mask=valid)

            d, cg = bin_scan(kneed)

            def pb(i, c):
                coffv, woffv = c
                ik = candk[pl.ds(16 * i, 16)]
                ii = candi[pl.ds(16 * i, 16)]
                valid = (16 * i + iota) < cand_n
                b = jnp.where((ik >> (s + 8)) == p_hi,
                              (ik >> s) & 0xFF, jnp.int32(256))
                b = jnp.where(valid, b, jnp.int32(-1))
                mw = b > d
                mc = b == d
                mwi = mw.astype(jnp.int32)
                mci = mc.astype(jnp.int32)
                csw = lax.cumsum(mwi, axis=0)
                wpos = woffv + csw - mwi
                plsc.store_scatter(wink, [wpos], ik, mask=mw)
                plsc.store_scatter(wini, [wpos], ii, mask=mw)
                csc = lax.cumsum(mci, axis=0)
                cpos = coffv + csc - mci
                plsc.store_scatter(candk, [cpos], ik, mask=mc)
                plsc.store_scatter(candi, [cpos], ii, mask=mc)
                return coffv + lastlane(csc), woffv + lastlane(csw)
            coffv, woffv = lax.fori_loop(
                0, nv, pb, (jnp.zeros((16,), jnp.int32), woffv))

            return coffv[0], kneed - cg, (p_hi << 8) | d, woffv

        cand_n, kneed, _p_hi, woff = lax.fori_loop(
            0, 3, do_round,
            (cand_n, jnp.int32(K), d0s, jnp.zeros((16,), jnp.int32)))

        def tl(i, _):
            ik = candk[pl.ds(16 * i, 16)]
            ii = candi[pl.ds(16 * i, 16)]
            e = 16 * i + iota
            m = e < kneed
            plsc.store_scatter(wink, [woff + e], ik, mask=m)
            plsc.store_scatter(wini, [woff + e], ii, mask=m)
            return 0
        lax.fori_loop(0, (kneed + 15) // 16, tl, 0)

        ks = [wink[pl.ds(16 * v, 16)] for v in range(KV)]
        iss = [wini[pl.ds(16 * v, 16)] for v in range(KV)]
        for kk in [2, 4, 8, 16, 32, 64, 128, 256]:
            jj = kk // 2
            while jj >= 1:
                if jj >= 16:
                    jv = jj // 16
                    for v in range(KV):
                        p = v ^ jv
                        if p > v:
                            ak, bk = ks[v], ks[p]
                            ai, bi = iss[v], iss[p]
                            c = (ak > bk) | ((ak == bk) & (ai < bi))
                            asc = ((16 * v) & kk) == 0
                            swap = ~c if asc else c
                            ks[v] = jnp.where(swap, bk, ak)
                            ks[p] = jnp.where(swap, ak, bk)
                            iss[v] = jnp.where(swap, bi, ai)
                            iss[p] = jnp.where(swap, ai, bi)
                else:
                    perm2d = (iota ^ jj).reshape(16, 1)
                    is_l = (iota & jj) != 0
                    for v in range(KV):
                        ak, ai = ks[v], iss[v]
                        pk = _perm(ak, perm2d)
                        pi = _perm(ai, perm2d)
                        c = (ak > pk) | ((ak == pk) & (ai < pi))
                        ascv = ((iota + 16 * v) & kk) == 0
                        keep = jnp.logical_xor(
                            jnp.logical_xor(c, is_l), ~ascv)
                        ks[v] = jnp.where(keep, ak, pk)
                        iss[v] = jnp.where(keep, ai, pi)
                jj //= 2

        for v in range(KV):
            obufv[pl.ds(16 * v, 16)] = lax.bitcast_convert_type(
                ks[v] ^ ((ks[v] >> 31) & jnp.int32(0x7FFFFFFF)), jnp.float32)
            obufi[pl.ds(16 * v, 16)] = iss[v]

        pltpu.sync_copy(obufv, vals_hbm.at[row])
        pltpu.sync_copy(obufi, idxs_hbm.at[row])
        cp.wait()
        return 0

    pltpu.sync_copy(logits_hbm.at[wid * ROWS_PER_W], rowbuf)
    lax.fori_loop(0, ROWS_PER_W, do_row, 0)


def kernel(logits):
    mesh = plsc.VectorSubcoreMesh(core_axis_name="c", subcore_axis_name="s")
    f = functools.partial(
        pl.kernel,
        out_type=[
            jax.ShapeDtypeStruct((ROWS, K), jnp.float32),
            jax.ShapeDtypeStruct((ROWS, K), jnp.int32),
        ],
        mesh=mesh,
        compiler_params=pltpu.CompilerParams(needs_layout_passes=False),
        scratch_types=[
            pltpu.VMEM((COLS,), jnp.float32),
            pltpu.VMEM((COLS,), jnp.int32),
            pltpu.VMEM((COLS,), jnp.int32),
            pltpu.VMEM((16 * 273 + 16,), jnp.int32),
            pltpu.VMEM((K,), jnp.int32),
            pltpu.VMEM((K,), jnp.int32),
            pltpu.VMEM((K,), jnp.float32),
            pltpu.VMEM((K,), jnp.int32),
            pltpu.SemaphoreType.DMA,
        ],
    )(_body)
    vals, idxs = f(logits)
    return vals, idxs


# speedup vs baseline: 1.0893x; 1.0893x over previous
import functools

import numpy as np
import jax
import jax.numpy as jnp
from jax import lax
from jax.experimental import pallas as pl
from jax.experimental.pallas import tpu as pltpu
from jax.experimental.pallas import tpu_sc as plsc

K = 256
ROWS = 128
COLS = 32768
NW = 32
ROWS_PER_W = ROWS // NW
NVREG_ROW = COLS // 16
KV = K // 16

_GDN = lax.GatherDimensionNumbers(
    offset_dims=(), collapsed_slice_dims=(0,), start_index_map=(0,)
)


def _perm(x, idx2d):
    return lax.gather(
        x, idx2d, _GDN, (1,),
        mode=lax.GatherScatterMode.PROMISE_IN_BOUNDS,
    )


def _key(x):
    i = lax.bitcast_convert_type(x, jnp.int32)
    return i ^ ((i >> 31) & jnp.int32(0x7FFFFFFF))


def _body(logits_hbm, vals_hbm, idxs_hbm,
          rowbuf, candk, candi, hist, wink, wini, obufv, obufi, dmasem):
    wid = lax.axis_index("s") * 2 + lax.axis_index("c")
    iota = lax.iota(jnp.int32, 16)
    ones = jnp.ones((16,), jnp.int32)
    zeros = jnp.zeros((16,), jnp.int32)
    lane_off = iota * 273
    lane_off128 = iota * 273 + 128
    idx15 = ((iota & 0) + 15).reshape(16, 1)

    def lastlane(x):
        return _perm(x, idx15)

    def clear_hist():
        @plsc.parallel_loop(0, 18, unroll=2)
        def cb(j):
            for l in range(16):
                hist[pl.ds(l * 273 + 16 * j, 16)] = zeros

    def bin_scan(kneed):
        hv = hist[pl.ds(256, 16)]
        for l in range(1, 16):
            hv = hv + hist[pl.ds(l * 273 + 256, 16)]
        cum = jnp.sum(hv)
        d_star = jnp.int32(256)
        cnt_gt = jnp.int32(0)
        found = jnp.int32(0)
        for j in range(15, -1, -1):
            totv = hist[pl.ds(16 * j, 16)]
            for l in range(1, 16):
                totv = totv + hist[pl.ds(l * 273 + 16 * j, 16)]
            sfx = lax.rev(lax.cumsum(lax.rev(totv, (0,)), axis=0), (0,)) + cum
            mask = sfx >= kneed
            popc = jnp.sum(mask.astype(jnp.int32))
            anyhit = (popc > 0).astype(jnp.int32)
            blane = popc - 1
            sfx_at = jnp.min(jnp.where(mask, sfx, jnp.int32(2**30)))
            tot_at = jnp.sum(jnp.where(iota == blane, totv, 0))
            first = anyhit * (1 - found)
            d_star = jnp.where(first > 0, 16 * j + blane, d_star)
            cnt_gt = jnp.where(first > 0, sfx_at - tot_at, cnt_gt)
            found = jnp.maximum(found, anyhit)
            cum = jnp.max(sfx)
        return d_star, cnt_gt

    def do_row(r, _):
        row = wid * ROWS_PER_W + r

        clear_hist()

        @plsc.parallel_loop(0, NVREG_ROW, unroll=8)
        def h1(i):
            t = lax.bitcast_convert_type(
                rowbuf[pl.ds(16 * i, 16)], jnp.int32) >> 24
            b = t ^ ((t >> 8) & jnp.int32(0x7F))
            plsc.addupdate_scatter(hist, [lane_off128 + b], ones)

        d0, _cg0 = bin_scan(jnp.int32(K))
        d0s = d0 - 128

        @plsc.parallel_loop(0, NVREG_ROW, unroll=8,
                            carry=jnp.zeros((16,), jnp.int32))
        def p1(i, offv):
            ik = _key(rowbuf[pl.ds(16 * i, 16)])
            m = (ik >> 24) >= d0s
            mi = m.astype(jnp.int32)
            cs = lax.cumsum(mi, axis=0)
            pos = offv + cs - mi
            plsc.store_scatter(candk, [pos], ik, mask=m)
            plsc.store_scatter(candi, [pos], iota + 16 * i, mask=m)
            return offv + lastlane(cs)
        cand_n = p1[0]

        nxt = jnp.minimum(r + 1, ROWS_PER_W - 1)
        cp = pltpu.async_copy(
            logits_hbm.at[wid * ROWS_PER_W + nxt], rowbuf, dmasem)

        def do_round(t, carry):
            cand_n, kneed, p_hi, woffv = carry
            s = 16 - 8 * t
            clear_hist()
            nv = (cand_n + 15) // 16

            @plsc.parallel_loop(0, nv, unroll=4)
            def hb(i):
                ik = candk[pl.ds(16 * i, 16)]
                valid = (16 * i + iota) < cand_n
                b = jnp.where((ik >> (s + 8)) == p_hi,
                              (ik >> s) & 0xFF, jnp.int32(256))
                plsc.addupdate_scatter(hist, [lane_off + b], ones, mask=valid)

            d, cg = bin_scan(kneed)

            def pb(i, c):
                coffv, woffv = c
                ik = candk[pl.ds(16 * i, 16)]
                ii = candi[pl.ds(16 * i, 16)]
                valid = (16 * i + iota) < cand_n
                b = jnp.where((ik >> (s + 8)) == p_hi,
                              (ik >> s) & 0xFF, jnp.int32(256))
                b = jnp.where(valid, b, jnp.int32(-1))
                mw = b > d
                mc = b == d
                mwi = mw.astype(jnp.int32)
                mci = mc.astype(jnp.int32)
                csw = lax.cumsum(mwi, axis=0)
                wpos = woffv + csw - mwi
                plsc.store_scatter(wink, [wpos], ik, mask=mw)
                plsc.store_scatter(wini, [wpos], ii, mask=mw)
                csc = lax.cumsum(mci, axis=0)
                cpos = coffv + csc - mci
                plsc.store_scatter(candk, [cpos], ik, mask=mc)
                plsc.store_scatter(candi, [cpos], ii, mask=mc)
                return coffv + lastlane(csc), woffv + lastlane(csw)
            coffv, woffv = lax.fori_loop(
                0, nv, pb, (jnp.zeros((16,), jnp.int32), woffv))

            return coffv[0], kneed - cg, (p_hi << 8) | d, woffv

        cand_n, kneed, _p_hi, woff = lax.fori_loop(
            0, 3, do_round,
            (cand_n, jnp.int32(K), d0s, jnp.zeros((16,), jnp.int32)))

        def tl(i, _):
            ik = candk[pl.ds(16 * i, 16)]
            ii = candi[pl.ds(16 * i, 16)]
            e = 16 * i + iota
            m = e < kneed
            plsc.store_scatter(wink, [woff + e], ik, mask=m)
            plsc.store_scatter(wini, [woff + e], ii, mask=m)
            return 0
        lax.fori_loop(0, (kneed + 15) // 16, tl, 0)

        ks = [wink[pl.ds(16 * v, 16)] for v in range(KV)]
        iss = [wini[pl.ds(16 * v, 16)] for v in range(KV)]
        for kk in [2, 4, 8, 16, 32, 64, 128, 256]:
            jj = kk // 2
            while jj >= 1:
                if jj >= 16:
                    jv = jj // 16
                    for v in range(KV):
                        p = v ^ jv
                        if p > v:
                            ak, bk = ks[v], ks[p]
                            ai, bi = iss[v], iss[p]
                            c = (ak > bk) | ((ak == bk) & (ai < bi))
                            asc = ((16 * v) & kk) == 0
                            swap = ~c if asc else c
                            ks[v] = jnp.where(swap, bk, ak)
                            ks[p] = jnp.where(swap, ak, bk)
                            iss[v] = jnp.where(swap, bi, ai)
                            iss[p] = jnp.where(swap, ai, bi)
                else:
                    perm2d = (iota ^ jj).reshape(16, 1)
                    is_l = (iota & jj) != 0
                    for v in range(KV):
                        ak, ai = ks[v], iss[v]
                        pk = _perm(ak, perm2d)
                        pi = _perm(ai, perm2d)
                        c = (ak > pk) | ((ak == pk) & (ai < pi))
                        ascv = ((iota + 16 * v) & kk) == 0
                        keep = jnp.logical_xor(
                            jnp.logical_xor(c, is_l), ~ascv)
                        ks[v] = jnp.where(keep, ak, pk)
                        iss[v] = jnp.where(keep, ai, pi)
                jj //= 2

        for v in range(KV):
            obufv[pl.ds(16 * v, 16)] = lax.bitcast_convert_type(
                ks[v] ^ ((ks[v] >> 31) & jnp.int32(0x7FFFFFFF)), jnp.float32)
            obufi[pl.ds(16 * v, 16)] = iss[v]

        pltpu.sync_copy(obufv, vals_hbm.at[row])
        pltpu.sync_copy(obufi, idxs_hbm.at[row])
        cp.wait()
        return 0

    pltpu.sync_copy(logits_hbm.at[wid * ROWS_PER_W], rowbuf)
    lax.fori_loop(0, ROWS_PER_W, do_row, 0)


def kernel(logits):
    mesh = plsc.VectorSubcoreMesh(core_axis_name="c", subcore_axis_name="s")
    f = functools.partial(
        pl.kernel,
        out_type=[
            jax.ShapeDtypeStruct((ROWS, K), jnp.float32),
            jax.ShapeDtypeStruct((ROWS, K), jnp.int32),
        ],
        mesh=mesh,
        compiler_params=pltpu.CompilerParams(needs_layout_passes=False),
        scratch_types=[
            pltpu.VMEM((COLS,), jnp.float32),
            pltpu.VMEM((COLS,), jnp.int32),
            pltpu.VMEM((COLS,), jnp.int32),
            pltpu.VMEM((16 * 273 + 16,), jnp.int32),
            pltpu.VMEM((K,), jnp.int32),
            pltpu.VMEM((K,), jnp.int32),
            pltpu.VMEM((K,), jnp.float32),
            pltpu.VMEM((K,), jnp.int32),
            pltpu.SemaphoreType.DMA,
        ],
    )(_body)
    vals, idxs = f(logits)
    return vals, idxs


# speedup vs baseline: 1.0897x; 1.0004x over previous
import functools

import jax
import jax.numpy as jnp
from jax import lax
from jax.experimental import pallas as pl
from jax.experimental.pallas import tpu as pltpu
from jax.experimental.pallas import tpu_sc as plsc

K = 256
ROWS = 128
COLS = 32768
NW = 32
ROWS_PER_W = ROWS // NW
NVREG_ROW = COLS // 16
KV = K // 16

_GDN = lax.GatherDimensionNumbers(
    offset_dims=(), collapsed_slice_dims=(0,), start_index_map=(0,)
)


def _perm(x, idx2d):
    return lax.gather(
        x, idx2d, _GDN, (1,),
        mode=lax.GatherScatterMode.PROMISE_IN_BOUNDS,
    )


def _key(x):
    i = lax.bitcast_convert_type(x, jnp.int32)
    return i ^ ((i >> 31) & jnp.int32(0x7FFFFFFF))


def _body(logits_hbm, vals_hbm, idxs_hbm,
          rowbuf, candk, candi, hist, wink, wini, obufv, obufi, dmasem):
    wid = lax.axis_index("s") * 2 + lax.axis_index("c")
    iota = lax.iota(jnp.int32, 16)
    ones = jnp.ones((16,), jnp.int32)
    zeros = jnp.zeros((16,), jnp.int32)
    lane_off = iota * 273
    lane_off128 = iota * 273 + 128
    idx15 = ((iota & 0) + 15).reshape(16, 1)

    def lastlane(x):
        return _perm(x, idx15)

    def clear_hist():
        @plsc.parallel_loop(0, 18, unroll=2)
        def cb(j):
            for l in range(16):
                hist[pl.ds(l * 273 + 16 * j, 16)] = zeros

    def bin_scan(kneed):
        hv = hist[pl.ds(256, 16)]
        for l in range(1, 16):
            hv = hv + hist[pl.ds(l * 273 + 256, 16)]
        cum = jnp.sum(hv)
        d_star = jnp.int32(256)
        cnt_gt = jnp.int32(0)
        found = jnp.int32(0)
        for j in range(15, -1, -1):
            totv = hist[pl.ds(16 * j, 16)]
            for l in range(1, 16):
                totv = totv + hist[pl.ds(l * 273 + 16 * j, 16)]
            sfx = lax.rev(lax.cumsum(lax.rev(totv, (0,)), axis=0), (0,)) + cum
            mask = sfx >= kneed
            popc = jnp.sum(mask.astype(jnp.int32))
            anyhit = (popc > 0).astype(jnp.int32)
            blane = popc - 1
            sfx_at = jnp.min(jnp.where(mask, sfx, jnp.int32(2**30)))
            tot_at = jnp.sum(jnp.where(iota == blane, totv, 0))
            first = anyhit * (1 - found)
            d_star = jnp.where(first > 0, 16 * j + blane, d_star)
            cnt_gt = jnp.where(first > 0, sfx_at - tot_at, cnt_gt)
            found = jnp.maximum(found, anyhit)
            cum = jnp.max(sfx)
        return d_star, cnt_gt

    def do_row(r, _):
        row = wid * ROWS_PER_W + r

        clear_hist()

        @plsc.parallel_loop(0, NVREG_ROW, unroll=8)
        def h1(i):
            t = lax.bitcast_convert_type(
                rowbuf[pl.ds(16 * i, 16)], jnp.int32) >> 24
            b = t ^ ((t >> 8) & jnp.int32(0x7F))
            plsc.addupdate_scatter(hist, [lane_off128 + b], ones)

        d0, _cg0 = bin_scan(jnp.int32(K))
        d0s = d0 - 128

        @plsc.parallel_loop(0, NVREG_ROW, unroll=8,
                            carry=jnp.zeros((16,), jnp.int32))
        def p1(i, offv):
            ik = _key(rowbuf[pl.ds(16 * i, 16)])
            m = (ik >> 24) >= d0s
            mi = m.astype(jnp.int32)
            cs = lax.cumsum(mi, axis=0)
            pos = offv + cs - mi
            plsc.store_scatter(candk, [pos], ik, mask=m)
            plsc.store_scatter(candi, [pos], iota + 16 * i, mask=m)
            return offv + lastlane(cs)
        cand_n = p1[0]

        nxt = jnp.minimum(r + 1, ROWS_PER_W - 1)
        cp = pltpu.async_copy(
            logits_hbm.at[wid * ROWS_PER_W + nxt], rowbuf, dmasem)

        def do_round(t, carry):
            cand_n, kneed, p_hi, woffv = carry
            s = 16 - 8 * t
            clear_hist()
            nv = (cand_n + 15) // 16

            @plsc.parallel_loop(0, nv, unroll=4)
            def hb(i):
                ik = candk[pl.ds(16 * i, 16)]
                valid = (16 * i + iota) < cand_n
                b = jnp.where((ik >> (s + 8)) == p_hi,
                              (ik >> s) & 0xFF, jnp.int32(256))
                plsc.addupdate_scatter(hist, [lane_off + b], ones, mask=valid)

            d, cg = bin_scan(kneed)

            def pb(i, c):
                coffv, woffv = c
                ik = candk[pl.ds(16 * i, 16)]
                ii = candi[pl.ds(16 * i, 16)]
                valid = (16 * i + iota) < cand_n
                b = jnp.where((ik >> (s + 8)) == p_hi,
                              (ik >> s) & 0xFF, jnp.int32(256))
                b = jnp.where(valid, b, jnp.int32(-1))
                mw = b > d
                mc = b == d
                mwi = mw.astype(jnp.int32)
                mci = mc.astype(jnp.int32)
                csw = lax.cumsum(mwi, axis=0)
                wpos = woffv + csw - mwi
                plsc.store_scatter(wink, [wpos], ik, mask=mw)
                plsc.store_scatter(wini, [wpos], ii, mask=mw)
                csc = lax.cumsum(mci, axis=0)
                cpos = coffv + csc - mci
                plsc.store_scatter(candk, [cpos], ik, mask=mc)
                plsc.store_scatter(candi, [cpos], ii, mask=mc)
                return coffv + lastlane(csc), woffv + lastlane(csw)
            coffv, woffv = lax.fori_loop(
                0, nv, pb, (jnp.zeros((16,), jnp.int32), woffv))

            return coffv[0], kneed - cg, (p_hi << 8) | d, woffv

        cand_n, kneed, _p_hi, woff = lax.fori_loop(
            0, 3, do_round,
            (cand_n, jnp.int32(K), d0s, jnp.zeros((16,), jnp.int32)))

        def tl(i, _):
            ik = candk[pl.ds(16 * i, 16)]
            ii = candi[pl.ds(16 * i, 16)]
            e = 16 * i + iota
            m = e < kneed
            plsc.store_scatter(wink, [woff + e], ik, mask=m)
            plsc.store_scatter(wini, [woff + e], ii, mask=m)
            return 0
        lax.fori_loop(0, (kneed + 15) // 16, tl, 0)

        ks = [wink[pl.ds(16 * v, 16)] for v in range(KV)]
        iss = [wini[pl.ds(16 * v, 16)] for v in range(KV)]
        for kk in [2, 4, 8, 16, 32, 64, 128, 256]:
            jj = kk // 2
            while jj >= 1:
                if jj >= 16:
                    jv = jj // 16
                    for v in range(KV):
                        p = v ^ jv
                        if p > v:
                            ak, bk = ks[v], ks[p]
                            ai, bi = iss[v], iss[p]
                            c = (ak > bk) | ((ak == bk) & (ai < bi))
                            asc = ((16 * v) & kk) == 0
                            swap = ~c if asc else c
                            ks[v] = jnp.where(swap, bk, ak)
                            ks[p] = jnp.where(swap, ak, bk)
                            iss[v] = jnp.where(swap, bi, ai)
                            iss[p] = jnp.where(swap, ai, bi)
                else:
                    perm2d = (iota ^ jj).reshape(16, 1)
                    is_l = (iota & jj) != 0
                    for v in range(KV):
                        ak, ai = ks[v], iss[v]
                        pk = _perm(ak, perm2d)
                        pi = _perm(ai, perm2d)
                        c = (ak > pk) | ((ak == pk) & (ai < pi))
                        ascv = ((iota + 16 * v) & kk) == 0
                        keep = jnp.logical_xor(
                            jnp.logical_xor(c, is_l), ~ascv)
                        ks[v] = jnp.where(keep, ak, pk)
                        iss[v] = jnp.where(keep, ai, pi)
                jj //= 2

        for v in range(KV):
            obufv[pl.ds(16 * v, 16)] = lax.bitcast_convert_type(
                ks[v] ^ ((ks[v] >> 31) & jnp.int32(0x7FFFFFFF)), jnp.float32)
            obufi[pl.ds(16 * v, 16)] = iss[v]

        pltpu.sync_copy(obufv, vals_hbm.at[row])
        pltpu.sync_copy(obufi, idxs_hbm.at[row])
        cp.wait()
        return 0

    pltpu.sync_copy(logits_hbm.at[wid * ROWS_PER_W], rowbuf)
    lax.fori_loop(0, ROWS_PER_W, do_row, 0)


def kernel(logits):
    mesh = plsc.VectorSubcoreMesh(core_axis_name="c", subcore_axis_name="s")
    f = functools.partial(
        pl.kernel,
        out_type=[
            jax.ShapeDtypeStruct((ROWS, K), jnp.float32),
            jax.ShapeDtypeStruct((ROWS, K), jnp.int32),
        ],
        mesh=mesh,
        compiler_params=pltpu.CompilerParams(needs_layout_passes=False),
        scratch_types=[
            pltpu.VMEM((COLS,), jnp.float32),
            pltpu.VMEM((COLS,), jnp.int32),
            pltpu.VMEM((COLS,), jnp.int32),
            pltpu.VMEM((16 * 273 + 16,), jnp.int32),
            pltpu.VMEM((K,), jnp.int32),
            pltpu.VMEM((K,), jnp.int32),
            pltpu.VMEM((K,), jnp.float32),
            pltpu.VMEM((K,), jnp.int32),
            pltpu.SemaphoreType.DMA,
        ],
    )(_body)
    vals, idxs = f(logits)
    return vals, idxs
